# Initial kernel scaffold; baseline (speedup 1.0000x reference)
import jax
import jax.numpy as jnp
from jax.experimental import pallas as pl


def kernel(x, W, b):
    raise NotImplementedError("write your pallas kernel here")


# speedup vs baseline: 1.6722x; 1.6722x over previous
import functools

import jax
import jax.numpy as jnp
from jax.experimental import pallas as pl
from jax.experimental.pallas import tpu as pltpu

_NUM_CLASSES = 285
_PAD_CLASSES = 384
_BIG = 2**30


def _tc_body(x_ref, w_ref, b_ref, o_ref, hist_ref, *, n_steps):
    i = pl.program_id(0)

    logits = jnp.dot(x_ref[...], w_ref[...],
                     preferred_element_type=jnp.float32) + b_ref[...]
    tt = logits.shape[0]
    cls = jax.lax.broadcasted_iota(jnp.int32, (tt, _PAD_CLASSES), 1)
    m = jnp.max(logits, axis=1, keepdims=True)
    votes = jnp.min(jnp.where(logits == m, cls, _BIG), axis=1, keepdims=True)
    counts = jnp.sum((votes == cls).astype(jnp.float32), axis=0, keepdims=True)

    @pl.when(i == 0)
    def _():
        hist_ref[...] = jnp.zeros_like(hist_ref)

    hist_ref[...] += counts

    @pl.when(i == n_steps - 1)
    def _():
        hist = hist_ref[...]
        bins = jax.lax.broadcasted_iota(jnp.int32, (1, _PAD_CLASSES), 1)
        hm = jnp.max(hist, axis=1, keepdims=True)
        winner = jnp.min(jnp.where(hist == hm, bins, _BIG), axis=1,
                         keepdims=True)
        o_ref[...] = (bins == winner).astype(jnp.float32)


def kernel(x, W, b):
    _, T, C = x.shape
    xr = x[0]
    w_pad = jnp.zeros((C, _PAD_CLASSES), jnp.float32).at[:, :_NUM_CLASSES].set(W.T)
    b_pad = jnp.full((1, _PAD_CLASSES), -3.4e38, jnp.float32).at[0, :_NUM_CLASSES].set(b)

    tt = 2048
    n_steps = T // tt
    out = pl.pallas_call(
        functools.partial(_tc_body, n_steps=n_steps),
        grid=(n_steps,),
        in_specs=[
            pl.BlockSpec((tt, C), lambda i: (i, 0)),
            pl.BlockSpec((C, _PAD_CLASSES), lambda i: (0, 0)),
            pl.BlockSpec((1, _PAD_CLASSES), lambda i: (0, 0)),
        ],
        out_specs=pl.BlockSpec((1, _PAD_CLASSES), lambda i: (0, 0)),
        out_shape=jax.ShapeDtypeStruct((1, _PAD_CLASSES), jnp.float32),
        scratch_shapes=[pltpu.VMEM((1, _PAD_CLASSES), jnp.float32)],
    )(xr, w_pad, b_pad)
    return out[:, :_NUM_CLASSES]
